# baseline (speedup 1.0000x reference)
import functools

import jax
import jax.numpy as jnp
from jax import lax
from jax.experimental import pallas as pl
from jax.experimental.pallas import tpu as pltpu
from jax.experimental.pallas import tpu_sc as plsc

N = 10000
E = 320000
F = 128
R = 20

_GDN = lax.GatherDimensionNumbers(
    offset_dims=(), collapsed_slice_dims=(0,), start_index_map=(0,))


def _bcast_lane(vec16, lane):
    idx = jnp.full((16, 1), lane, jnp.int32)
    return lax.gather(vec16, idx, _GDN, (1,),
                      mode=lax.GatherScatterMode.PROMISE_IN_BOUNDS)


NC = 2
NS = 16
NW = NC * NS
EPW = E // NW
B = 40
NCH = EPW // B
NP = 10240
STRIPE = NP // NS

EB = 2000
NB = 1000


def _phi_body(rbf_ref, fw1_ref, fb1_ref, fw2_ref, fb2_ref,
              ss_ref, vv_ref, sv_ref):
    h = jnp.dot(rbf_ref[...], fw1_ref[...],
                preferred_element_type=jnp.float32) + fb1_ref[...]
    h = h * lax.logistic(h)
    phi = jnp.dot(h, fw2_ref[...],
                  preferred_element_type=jnp.float32) + fb2_ref[...]
    ss_ref[...] = phi[:, 0:F]
    vv_ref[...] = phi[:, F:2 * F]
    sv_ref[...] = phi[:, 2 * F:3 * F]


def _phi_call(rbf, fw1, fb1, fw2, fb2):
    grid = (E // EB,)
    out = pl.pallas_call(
        _phi_body,
        grid=grid,
        in_specs=[
            pl.BlockSpec((EB, R), lambda i: (i, 0)),
            pl.BlockSpec((R, F), lambda i: (0, 0)),
            pl.BlockSpec((1, F), lambda i: (0, 0)),
            pl.BlockSpec((F, 3 * F), lambda i: (0, 0)),
            pl.BlockSpec((1, 3 * F), lambda i: (0, 0)),
        ],
        out_specs=[
            pl.BlockSpec((EB, F), lambda i: (i, 0)),
            pl.BlockSpec((EB, F), lambda i: (i, 0)),
            pl.BlockSpec((EB, F), lambda i: (i, 0)),
        ],
        out_shape=[jax.ShapeDtypeStruct((E, F), jnp.float32)] * 3,
    )(rbf, fw1, fb1, fw2, fb2)
    return out


def _edge_body(row_h, col_h, stab_h, vtab_h,
               pss_h, pvv_h, psv_h, dall_h, zeros_h,
               ms_out, mv_out, w_out,
               col_v, row_v, dir_v, phiA, phiB, gath, mbuf, wbuf, acc,
               sem, semg0, semg1, sems0, sems1, seml0, seml1, semr0, semr1):
    cidx = lax.axis_index("c")
    sidx = lax.axis_index("s")
    wid = sidx * NC + cidx
    ebase0 = wid * EPW
    stripe = sidx * STRIPE
    semg = (semg0, semg1)
    sems = (sems0, sems1)
    seml = (seml0, seml1)
    semr = (semr0, semr1)

    def fire_lin_s(j, b):
        base = ebase0 + j * B
        pltpu.async_copy(col_h.at[pl.ds(base, B)], col_v.at[b], seml[b])
        pltpu.async_copy(pss_h.at[pl.ds(base, B)], phiA.at[b], seml[b])
        pltpu.async_copy(psv_h.at[pl.ds(base, B)], phiB.at[b], seml[b])

    def wait_lin_s(j, b):
        base = ebase0 + j * B
        pltpu.make_async_copy(col_h.at[pl.ds(base, B)], col_v.at[b],
                              seml[b]).wait()
        pltpu.make_async_copy(pss_h.at[pl.ds(base, B)], phiA.at[b],
                              seml[b]).wait()
        pltpu.make_async_copy(psv_h.at[pl.ds(base, B)], phiB.at[b],
                              seml[b]).wait()

    def fire_gath_s(b):
        pltpu.async_copy(stab_h.at[col_v.at[b]], gath.at[b], semg[b])

    def wait_gath_s(b):
        pltpu.make_async_copy(stab_h.at[col_v.at[b]], gath.at[b],
                              semg[b]).wait()

    def fire_row_s(j, b):
        base = ebase0 + j * B
        pltpu.async_copy(row_h.at[pl.ds(base, B)], row_v.at[b], semr[b])

    def wait_row_s(j, b):
        base = ebase0 + j * B
        pltpu.make_async_copy(row_h.at[pl.ds(base, B)], row_v.at[b],
                              semr[b]).wait()

    def fire_scat_s(b):
        pltpu.async_copy(mbuf.at[b], acc.at[row_v.at[b]], sems[b],
                         add=True)

    def wait_scat_s(b):
        pltpu.make_async_copy(mbuf.at[b], acc.at[row_v.at[b]],
                              sems[b]).wait()

    def compute_s(b):
        @plsc.parallel_loop(0, B, unroll=4)
        def _(e):
            for j in range(F // 16):
                sl = pl.ds(j * 16, 16)
                sv = gath[b, e, sl]
                wbuf[e, sl] = phiB[b, e, sl] * sv
                mbuf[b, e, sl] = phiA[b, e, sl] * sv

    pltpu.sync_copy(zeros_h.at[pl.ds(stripe, STRIPE)],
                    acc.at[pl.ds(stripe, STRIPE)])
    plsc.subcore_barrier()

    fire_lin_s(0, 0)
    fire_lin_s(1, 1)
    wait_lin_s(0, 0)
    fire_gath_s(0)

    def half_s(b, j_b):
        base = ebase0 + j_b * B
        wait_gath_s(b)

        @pl.when(j_b >= 2)
        def _():
            wait_scat_s(b)
        fire_row_s(j_b, b)
        compute_s(b)
        pltpu.sync_copy(wbuf, w_out.at[pl.ds(base, B)])
        wait_row_s(j_b, b)
        fire_scat_s(b)

        @pl.when(j_b + 2 < NCH)
        def _():
            fire_lin_s(j_b + 2, b)

        @pl.when(j_b + 1 < NCH)
        def _():
            wait_lin_s(j_b + 1, 1 - b)
            fire_gath_s(1 - b)

    def step_s(g, carry):
        j = g * 2
        half_s(0, j)
        half_s(1, j + 1)
        return carry

    lax.fori_loop(0, NCH // 2, step_s, 0)
    wait_scat_s(0)
    wait_scat_s(1)
    plsc.subcore_barrier()
    pltpu.sync_copy(acc.at[pl.ds(stripe, STRIPE)],
                    ms_out.at[pl.ds(cidx * NP + stripe, STRIPE)])

    def fire_lin(j, b, p):
        base = ebase0 + j * B
        pltpu.async_copy(col_h.at[pl.ds(base, B)], col_v.at[b], seml[b])
        pltpu.async_copy(pvv_h.at[pl.ds(base, B)], phiA.at[b], seml[b])
        pltpu.async_copy(w_out.at[pl.ds(base, B)], phiB.at[b], seml[b])
        pltpu.async_copy(dall_h.at[pl.ds(p * E + base, B)], dir_v.at[b],
                         seml[b])

    def wait_lin(j, b, p):
        base = ebase0 + j * B
        pltpu.make_async_copy(col_h.at[pl.ds(base, B)], col_v.at[b],
                              seml[b]).wait()
        pltpu.make_async_copy(pvv_h.at[pl.ds(base, B)], phiA.at[b],
                              seml[b]).wait()
        pltpu.make_async_copy(w_out.at[pl.ds(base, B)], phiB.at[b],
                              seml[b]).wait()
        pltpu.make_async_copy(dall_h.at[pl.ds(p * E + base, B)],
                              dir_v.at[b], seml[b]).wait()

    def adjust_col(b, p):
        poff = p * N
        for st in (0, 16):
            sl = pl.ds(st, 16)
            col_v[b, sl] = col_v[b, sl] + poff
        lane = lax.iota(jnp.int32, 16)
        sl = pl.ds(24, 16)
        col_v[b, sl] = col_v[b, sl] + jnp.where(lane >= 8, poff, 0)

    def fire_gath(b):
        pltpu.async_copy(vtab_h.at[col_v.at[b]], gath.at[b], semg[b])

    def wait_gath(b):
        pltpu.make_async_copy(vtab_h.at[col_v.at[b]], gath.at[b],
                              semg[b]).wait()

    def fire_row(j, b):
        base = ebase0 + j * B
        pltpu.async_copy(row_h.at[pl.ds(base, B)], row_v.at[b], semr[b])

    def wait_row(j, b):
        base = ebase0 + j * B
        pltpu.make_async_copy(row_h.at[pl.ds(base, B)], row_v.at[b],
                              semr[b]).wait()

    def fire_scat(b):
        pltpu.async_copy(mbuf.at[b], acc.at[row_v.at[b]], sems[b],
                         add=True)

    def wait_scat(b):
        pltpu.make_async_copy(mbuf.at[b], acc.at[row_v.at[b]],
                              sems[b]).wait()

    def compute_v(b):
        for g0 in (0, 16, 24):
            dv = dir_v[b, pl.ds(g0, 16)]

            @plsc.parallel_loop(0, 16, unroll=6)
            def _(e16):
                e = g0 + e16
                de = _bcast_lane(dv, e16)
                for j in range(F // 16):
                    sl = pl.ds(j * 16, 16)
                    mbuf[b, e, sl] = (phiA[b, e, sl] * gath[b, e, sl]
                                      + de * phiB[b, e, sl])

    def vpass(p, carry0):
        pltpu.sync_copy(zeros_h.at[pl.ds(stripe, STRIPE)],
                        acc.at[pl.ds(stripe, STRIPE)])
        plsc.subcore_barrier()

        fire_lin(0, 0, p)
        fire_lin(1, 1, p)
        wait_lin(0, 0, p)
        adjust_col(0, p)
        fire_gath(0)

        def half(b, j_b):
            wait_gath(b)

            @pl.when(j_b >= 2)
            def _():
                wait_scat(b)
            fire_row(j_b, b)
            compute_v(b)
            wait_row(j_b, b)
            fire_scat(b)

            @pl.when(j_b + 2 < NCH)
            def _():
                fire_lin(j_b + 2, b, p)

            @pl.when(j_b + 1 < NCH)
            def _():
                wait_lin(j_b + 1, 1 - b, p)
                adjust_col(1 - b, p)
                fire_gath(1 - b)

        def step(g, carry):
            j = g * 2
            half(0, j)
            half(1, j + 1)
            return carry

        lax.fori_loop(0, NCH // 2, step, 0)
        wait_scat(0)
        wait_scat(1)
        plsc.subcore_barrier()
        pltpu.sync_copy(
            acc.at[pl.ds(stripe, STRIPE)],
            mv_out.at[pl.ds(p * (2 * NP) + cidx * NP + stripe, STRIPE)])
        plsc.subcore_barrier()
        return carry0

    lax.fori_loop(0, 3, vpass, 0)


def _edge_call(row, col, s, vtab, pss, pvv, psv, dall, zeros):
    mesh = plsc.VectorSubcoreMesh(core_axis_name="c", subcore_axis_name="s")
    fn = functools.partial(
        pl.kernel,
        mesh=mesh,
        out_type=[
            jax.ShapeDtypeStruct((2 * NP, F), jnp.float32),
            jax.ShapeDtypeStruct((6 * NP, F), jnp.float32),
            jax.ShapeDtypeStruct((E, F), jnp.float32),
        ],
        scratch_types=[
            pltpu.VMEM((2, B), jnp.int32),
            pltpu.VMEM((2, B), jnp.int32),
            pltpu.VMEM((2, B), jnp.float32),
            pltpu.VMEM((2, B, F), jnp.float32),
            pltpu.VMEM((2, B, F), jnp.float32),
            pltpu.VMEM((2, B, F), jnp.float32),
            pltpu.VMEM((2, B, F), jnp.float32),
            pltpu.VMEM((B, F), jnp.float32),
            pltpu.VMEM_SHARED((NP, F), jnp.float32),
            pltpu.SemaphoreType.DMA,
            pltpu.SemaphoreType.DMA,
            pltpu.SemaphoreType.DMA,
            pltpu.SemaphoreType.DMA,
            pltpu.SemaphoreType.DMA,
            pltpu.SemaphoreType.DMA,
            pltpu.SemaphoreType.DMA,
            pltpu.SemaphoreType.DMA,
            pltpu.SemaphoreType.DMA,
        ],
    )(_edge_body)
    return fn(row, col, s, vtab, pss, pvv, psv, dall, zeros)


def _update_body(s_ref, msp_ref, mvp_ref, vt_ref,
                 uw1_ref, ub1_ref, uw2_ref, ub2_ref,
                 sout_ref, vtout_ref):
    m_s = msp_ref[0] + msp_ref[1]
    mv = mvp_ref[:, 0] + mvp_ref[:, 1]
    vn = jnp.sqrt(mv[0] * mv[0] + mv[1] * mv[1] + mv[2] * mv[2])
    u_in = jnp.concatenate([s_ref[...], m_s, vn], axis=-1)
    h = jnp.dot(u_in, uw1_ref[...],
                preferred_element_type=jnp.float32) + ub1_ref[...]
    h = h * lax.logistic(h)
    u = jnp.dot(h, uw2_ref[...],
                preferred_element_type=jnp.float32) + ub2_ref[...]
    sout_ref[...] = s_ref[...] + u[:, 0:F]
    alpha = u[:, F:2 * F]
    beta = u[:, 2 * F:3 * F]
    vtout_ref[...] = alpha[None] * vt_ref[...] + beta[None] * mv


def _update_call(s, msp, mvp, vt, uw1, ub1, uw2, ub2):
    grid = (N // NB,)
    return pl.pallas_call(
        _update_body,
        grid=grid,
        in_specs=[
            pl.BlockSpec((NB, F), lambda i: (i, 0)),
            pl.BlockSpec((2, NB, F), lambda i: (0, i, 0)),
            pl.BlockSpec((3, 2, NB, F), lambda i: (0, 0, i, 0)),
            pl.BlockSpec((3, NB, F), lambda i: (0, i, 0)),
            pl.BlockSpec((3 * F, F), lambda i: (0, 0)),
            pl.BlockSpec((1, F), lambda i: (0, 0)),
            pl.BlockSpec((F, 3 * F), lambda i: (0, 0)),
            pl.BlockSpec((1, 3 * F), lambda i: (0, 0)),
        ],
        out_specs=[
            pl.BlockSpec((NB, F), lambda i: (i, 0)),
            pl.BlockSpec((3, NB, F), lambda i: (0, i, 0)),
        ],
        out_shape=[
            jax.ShapeDtypeStruct((N, F), jnp.float32),
            jax.ShapeDtypeStruct((3, N, F), jnp.float32),
        ],
    )(s, msp, mvp, vt, uw1, ub1, uw2, ub2)


def kernel(s, v, edge_index, edge_attr, rbf, fw1, fb1, fw2, fb2,
           uw1, ub1, uw2, ub2):
    row = edge_index[0].astype(jnp.int32)
    col = edge_index[1].astype(jnp.int32)
    dirs = edge_attr[:, 1:4]
    vt = jnp.transpose(v, (2, 0, 1))

    pss, pvv, psv = _phi_call(rbf, fw1, fb1.reshape(1, F), fw2,
                              fb2.reshape(1, 3 * F))
    zeros = jnp.zeros((NP, F), jnp.float32)
    vtab = vt.reshape(3 * N, F)
    dall = jnp.transpose(dirs).reshape(3 * E)
    ms2, mv6, _ = _edge_call(row, col, s, vtab, pss, pvv, psv, dall, zeros)
    msp = ms2.reshape(2, NP, F)[:, :N]
    mvp = mv6.reshape(3, 2, NP, F)[:, :, :N]
    s_out, vt_out = _update_call(s, msp, mvp, vt, uw1, ub1.reshape(1, F),
                                 uw2, ub2.reshape(1, 3 * F))
    v_out = jnp.transpose(vt_out, (1, 2, 0))
    return (s_out, v_out)


# speedup vs baseline: 1.1907x; 1.1907x over previous
import functools

import jax
import jax.numpy as jnp
from jax import lax
from jax.experimental import pallas as pl
from jax.experimental.pallas import tpu as pltpu
from jax.experimental.pallas import tpu_sc as plsc

N = 10000
E = 320000
F = 128
R = 20

_GDN = lax.GatherDimensionNumbers(
    offset_dims=(), collapsed_slice_dims=(0,), start_index_map=(0,))


def _bcast_lane(vec16, lane):
    idx = jnp.full((16, 1), lane, jnp.int32)
    return lax.gather(vec16, idx, _GDN, (1,),
                      mode=lax.GatherScatterMode.PROMISE_IN_BOUNDS)


NC = 2
NS = 16
NW = NC * NS
EPW = E // NW
B = 40
NCH = EPW // B
NP = 10240
STRIPE = NP // NS

EB = 2000
NB = 1000


def _phi_body(rbf_ref, fw1_ref, fb1_ref, fw2_ref, fb2_ref,
              ss_ref, vv_ref, sv_ref):
    h = jnp.dot(rbf_ref[...], fw1_ref[...],
                preferred_element_type=jnp.float32) + fb1_ref[...]
    h = h * lax.logistic(h)
    phi = jnp.dot(h, fw2_ref[...],
                  preferred_element_type=jnp.float32) + fb2_ref[...]
    ss_ref[...] = phi[:, 0:F]
    vv_ref[...] = phi[:, F:2 * F]
    sv_ref[...] = phi[:, 2 * F:3 * F]


def _phi_call(rbf, fw1, fb1, fw2, fb2):
    grid = (E // EB,)
    out = pl.pallas_call(
        _phi_body,
        grid=grid,
        in_specs=[
            pl.BlockSpec((EB, R), lambda i: (i, 0)),
            pl.BlockSpec((R, F), lambda i: (0, 0)),
            pl.BlockSpec((1, F), lambda i: (0, 0)),
            pl.BlockSpec((F, 3 * F), lambda i: (0, 0)),
            pl.BlockSpec((1, 3 * F), lambda i: (0, 0)),
        ],
        out_specs=[
            pl.BlockSpec((EB, F), lambda i: (i, 0)),
            pl.BlockSpec((EB, F), lambda i: (i, 0)),
            pl.BlockSpec((EB, F), lambda i: (i, 0)),
        ],
        out_shape=[jax.ShapeDtypeStruct((E, F), jnp.float32)] * 3,
    )(rbf, fw1, fb1, fw2, fb2)
    return out


def _edge_body(row_h, col_h, stab_h, vtab_h,
               pss_h, pvv_h, psv_h, dall_h, zeros_h,
               ms_out, mv_out, w_out,
               col_v, row_v, dir_v, phiA, phiB, gath, mbuf, wbuf, acc,
               sem, semg0, semg1, sems0, sems1, seml0, seml1, semr0, semr1):
    cidx = lax.axis_index("c")
    sidx = lax.axis_index("s")
    wid = sidx * NC + cidx
    ebase0 = wid * EPW
    stripe = sidx * STRIPE
    semg = (semg0, semg1)
    sems = (sems0, sems1)
    seml = (seml0, seml1)
    semr = (semr0, semr1)

    def fire_lin_s(j, b):
        base = ebase0 + j * B
        pltpu.async_copy(col_h.at[pl.ds(base, B)], col_v.at[b], seml[b])
        pltpu.async_copy(pss_h.at[pl.ds(base, B)], phiA.at[b], seml[b])
        pltpu.async_copy(psv_h.at[pl.ds(base, B)], phiB.at[b], seml[b])

    def wait_lin_s(j, b):
        base = ebase0 + j * B
        pltpu.make_async_copy(col_h.at[pl.ds(base, B)], col_v.at[b],
                              seml[b]).wait()
        pltpu.make_async_copy(pss_h.at[pl.ds(base, B)], phiA.at[b],
                              seml[b]).wait()
        pltpu.make_async_copy(psv_h.at[pl.ds(base, B)], phiB.at[b],
                              seml[b]).wait()

    def fire_gath_s(b):
        pltpu.async_copy(stab_h.at[col_v.at[b]], gath.at[b], semg[b])

    def wait_gath_s(b):
        pltpu.make_async_copy(stab_h.at[col_v.at[b]], gath.at[b],
                              semg[b]).wait()

    def fire_row_s(j, b):
        base = ebase0 + j * B
        pltpu.async_copy(row_h.at[pl.ds(base, B)], row_v.at[b], semr[b])

    def wait_row_s(j, b):
        base = ebase0 + j * B
        pltpu.make_async_copy(row_h.at[pl.ds(base, B)], row_v.at[b],
                              semr[b]).wait()

    def fire_scat_s(b):
        pltpu.async_copy(mbuf.at[b], acc.at[row_v.at[b]], sems[b],
                         add=True)

    def wait_scat_s(b):
        pltpu.make_async_copy(mbuf.at[b], acc.at[row_v.at[b]],
                              sems[b]).wait()

    def compute_s(b):
        @plsc.parallel_loop(0, B, unroll=8)
        def _(e):
            for j in range(F // 16):
                sl = pl.ds(j * 16, 16)
                sv = gath[b, e, sl]
                wbuf[e, sl] = phiB[b, e, sl] * sv
                mbuf[b, e, sl] = phiA[b, e, sl] * sv

    pltpu.sync_copy(zeros_h.at[pl.ds(stripe, STRIPE)],
                    acc.at[pl.ds(stripe, STRIPE)])
    plsc.subcore_barrier()

    fire_lin_s(0, 0)
    fire_lin_s(1, 1)
    wait_lin_s(0, 0)
    fire_gath_s(0)

    def half_s(b, j_b):
        base = ebase0 + j_b * B
        wait_gath_s(b)

        @pl.when(j_b >= 2)
        def _():
            wait_scat_s(b)
        fire_row_s(j_b, b)
        compute_s(b)
        pltpu.sync_copy(wbuf, w_out.at[pl.ds(base, B)])
        wait_row_s(j_b, b)
        fire_scat_s(b)

        @pl.when(j_b + 2 < NCH)
        def _():
            fire_lin_s(j_b + 2, b)

        @pl.when(j_b + 1 < NCH)
        def _():
            wait_lin_s(j_b + 1, 1 - b)
            fire_gath_s(1 - b)

    def step_s(g, carry):
        j = g * 2
        half_s(0, j)
        half_s(1, j + 1)
        return carry

    lax.fori_loop(0, NCH // 2, step_s, 0)
    wait_scat_s(0)
    wait_scat_s(1)
    plsc.subcore_barrier()
    pltpu.sync_copy(acc.at[pl.ds(stripe, STRIPE)],
                    ms_out.at[pl.ds(cidx * NP + stripe, STRIPE)])

    def fire_lin(j, b, p):
        base = ebase0 + j * B
        pltpu.async_copy(col_h.at[pl.ds(base, B)], col_v.at[b], seml[b])
        pltpu.async_copy(pvv_h.at[pl.ds(base, B)], phiA.at[b], seml[b])
        pltpu.async_copy(w_out.at[pl.ds(base, B)], phiB.at[b], seml[b])
        pltpu.async_copy(dall_h.at[pl.ds(p * E + base, B)], dir_v.at[b],
                         seml[b])

    def wait_lin(j, b, p):
        base = ebase0 + j * B
        pltpu.make_async_copy(col_h.at[pl.ds(base, B)], col_v.at[b],
                              seml[b]).wait()
        pltpu.make_async_copy(pvv_h.at[pl.ds(base, B)], phiA.at[b],
                              seml[b]).wait()
        pltpu.make_async_copy(w_out.at[pl.ds(base, B)], phiB.at[b],
                              seml[b]).wait()
        pltpu.make_async_copy(dall_h.at[pl.ds(p * E + base, B)],
                              dir_v.at[b], seml[b]).wait()

    def adjust_col(b, p):
        poff = p * N
        for st in (0, 16):
            sl = pl.ds(st, 16)
            col_v[b, sl] = col_v[b, sl] + poff
        lane = lax.iota(jnp.int32, 16)
        sl = pl.ds(24, 16)
        col_v[b, sl] = col_v[b, sl] + jnp.where(lane >= 8, poff, 0)

    def fire_gath(b):
        pltpu.async_copy(vtab_h.at[col_v.at[b]], gath.at[b], semg[b])

    def wait_gath(b):
        pltpu.make_async_copy(vtab_h.at[col_v.at[b]], gath.at[b],
                              semg[b]).wait()

    def fire_row(j, b):
        base = ebase0 + j * B
        pltpu.async_copy(row_h.at[pl.ds(base, B)], row_v.at[b], semr[b])

    def wait_row(j, b):
        base = ebase0 + j * B
        pltpu.make_async_copy(row_h.at[pl.ds(base, B)], row_v.at[b],
                              semr[b]).wait()

    def fire_scat(b):
        pltpu.async_copy(mbuf.at[b], acc.at[row_v.at[b]], sems[b],
                         add=True)

    def wait_scat(b):
        pltpu.make_async_copy(mbuf.at[b], acc.at[row_v.at[b]],
                              sems[b]).wait()

    def compute_v(b):
        for g0 in (0, 16, 24):
            dv = dir_v[b, pl.ds(g0, 16)]

            @plsc.parallel_loop(0, 16, unroll=4)
            def _(e16):
                e = g0 + e16
                de = _bcast_lane(dv, e16)
                for j in range(F // 16):
                    sl = pl.ds(j * 16, 16)
                    mbuf[b, e, sl] = (phiA[b, e, sl] * gath[b, e, sl]
                                      + de * phiB[b, e, sl])

    def vpass(p, carry0):
        pltpu.sync_copy(zeros_h.at[pl.ds(stripe, STRIPE)],
                        acc.at[pl.ds(stripe, STRIPE)])
        plsc.subcore_barrier()

        fire_lin(0, 0, p)
        fire_lin(1, 1, p)
        wait_lin(0, 0, p)
        adjust_col(0, p)
        fire_gath(0)

        def half(b, j_b):
            wait_gath(b)

            @pl.when(j_b >= 2)
            def _():
                wait_scat(b)
            fire_row(j_b, b)
            compute_v(b)
            wait_row(j_b, b)
            fire_scat(b)

            @pl.when(j_b + 2 < NCH)
            def _():
                fire_lin(j_b + 2, b, p)

            @pl.when(j_b + 1 < NCH)
            def _():
                wait_lin(j_b + 1, 1 - b, p)
                adjust_col(1 - b, p)
                fire_gath(1 - b)

        def step(g, carry):
            j = g * 2
            half(0, j)
            half(1, j + 1)
            return carry

        lax.fori_loop(0, NCH // 2, step, 0)
        wait_scat(0)
        wait_scat(1)
        plsc.subcore_barrier()
        pltpu.sync_copy(
            acc.at[pl.ds(stripe, STRIPE)],
            mv_out.at[pl.ds(p * (2 * NP) + cidx * NP + stripe, STRIPE)])
        plsc.subcore_barrier()
        return carry0

    lax.fori_loop(0, 3, vpass, 0)


def _edge_call(row, col, s, vtab, pss, pvv, psv, dall, zeros):
    mesh = plsc.VectorSubcoreMesh(core_axis_name="c", subcore_axis_name="s")
    fn = functools.partial(
        pl.kernel,
        mesh=mesh,
        out_type=[
            jax.ShapeDtypeStruct((2 * NP, F), jnp.float32),
            jax.ShapeDtypeStruct((6 * NP, F), jnp.float32),
            jax.ShapeDtypeStruct((E, F), jnp.float32),
        ],
        scratch_types=[
            pltpu.VMEM((2, B), jnp.int32),
            pltpu.VMEM((2, B), jnp.int32),
            pltpu.VMEM((2, B), jnp.float32),
            pltpu.VMEM((2, B, F), jnp.float32),
            pltpu.VMEM((2, B, F), jnp.float32),
            pltpu.VMEM((2, B, F), jnp.float32),
            pltpu.VMEM((2, B, F), jnp.float32),
            pltpu.VMEM((B, F), jnp.float32),
            pltpu.VMEM_SHARED((NP, F), jnp.float32),
            pltpu.SemaphoreType.DMA,
            pltpu.SemaphoreType.DMA,
            pltpu.SemaphoreType.DMA,
            pltpu.SemaphoreType.DMA,
            pltpu.SemaphoreType.DMA,
            pltpu.SemaphoreType.DMA,
            pltpu.SemaphoreType.DMA,
            pltpu.SemaphoreType.DMA,
            pltpu.SemaphoreType.DMA,
        ],
    )(_edge_body)
    return fn(row, col, s, vtab, pss, pvv, psv, dall, zeros)


def _update_body(s_ref, msp_ref, mvp_ref, vt_ref,
                 uw1_ref, ub1_ref, uw2_ref, ub2_ref,
                 sout_ref, vtout_ref):
    m_s = msp_ref[0] + msp_ref[1]
    mv = mvp_ref[:, 0] + mvp_ref[:, 1]
    vn = jnp.sqrt(mv[0] * mv[0] + mv[1] * mv[1] + mv[2] * mv[2])
    u_in = jnp.concatenate([s_ref[...], m_s, vn], axis=-1)
    h = jnp.dot(u_in, uw1_ref[...],
                preferred_element_type=jnp.float32) + ub1_ref[...]
    h = h * lax.logistic(h)
    u = jnp.dot(h, uw2_ref[...],
                preferred_element_type=jnp.float32) + ub2_ref[...]
    sout_ref[...] = s_ref[...] + u[:, 0:F]
    alpha = u[:, F:2 * F]
    beta = u[:, 2 * F:3 * F]
    vtout_ref[...] = alpha[None] * vt_ref[...] + beta[None] * mv


def _update_call(s, msp, mvp, vt, uw1, ub1, uw2, ub2):
    grid = (N // NB,)
    return pl.pallas_call(
        _update_body,
        grid=grid,
        in_specs=[
            pl.BlockSpec((NB, F), lambda i: (i, 0)),
            pl.BlockSpec((2, NB, F), lambda i: (0, i, 0)),
            pl.BlockSpec((3, 2, NB, F), lambda i: (0, 0, i, 0)),
            pl.BlockSpec((3, NB, F), lambda i: (0, i, 0)),
            pl.BlockSpec((3 * F, F), lambda i: (0, 0)),
            pl.BlockSpec((1, F), lambda i: (0, 0)),
            pl.BlockSpec((F, 3 * F), lambda i: (0, 0)),
            pl.BlockSpec((1, 3 * F), lambda i: (0, 0)),
        ],
        out_specs=[
            pl.BlockSpec((NB, F), lambda i: (i, 0)),
            pl.BlockSpec((3, NB, F), lambda i: (0, i, 0)),
        ],
        out_shape=[
            jax.ShapeDtypeStruct((N, F), jnp.float32),
            jax.ShapeDtypeStruct((3, N, F), jnp.float32),
        ],
    )(s, msp, mvp, vt, uw1, ub1, uw2, ub2)


def kernel(s, v, edge_index, edge_attr, rbf, fw1, fb1, fw2, fb2,
           uw1, ub1, uw2, ub2):
    row = edge_index[0].astype(jnp.int32)
    col = edge_index[1].astype(jnp.int32)
    dirs = edge_attr[:, 1:4]
    vt = jnp.transpose(v, (2, 0, 1))

    pss, pvv, psv = _phi_call(rbf, fw1, fb1.reshape(1, F), fw2,
                              fb2.reshape(1, 3 * F))
    zeros = jnp.zeros((NP, F), jnp.float32)
    vtab = vt.reshape(3 * N, F)
    dall = jnp.transpose(dirs).reshape(3 * E)
    ms2, mv6, _ = _edge_call(row, col, s, vtab, pss, pvv, psv, dall, zeros)
    msp = ms2.reshape(2, NP, F)[:, :N]
    mvp = mv6.reshape(3, 2, NP, F)[:, :, :N]
    s_out, vt_out = _update_call(s, msp, mvp, vt, uw1, ub1.reshape(1, F),
                                 uw2, ub2.reshape(1, 3 * F))
    v_out = jnp.transpose(vt_out, (1, 2, 0))
    return (s_out, v_out)
